# Initial kernel scaffold; baseline (speedup 1.0000x reference)
import jax
import jax.numpy as jnp
from jax.experimental import pallas as pl


def kernel(merchant_id, user_id, category_id, mcc, country, W_merchant_id, W_user_id, W_category_id, W_mcc, W_country):
    raise NotImplementedError("write your pallas kernel here")


# speedup vs baseline: 4.0217x; 4.0217x over previous
import jax
import jax.numpy as jnp
from jax import lax
from jax.experimental import pallas as pl
from jax.experimental.pallas import tpu as pltpu
from jax.experimental.pallas import tpu_sc as plsc

B, L = 4096, 50
ROWS = B * L
G = 128
DIMS = (32, 32, 16, 16, 16)
OFFS = (0, 32, 64, 80, 96)
DTOT = 112

_info = plsc.get_sparse_core_info()
NC, NS = _info.num_cores, _info.num_subcores
NW = NC * NS
PERW = ROWS // NW
NG = PERW // G


def _sc_body(mid, uid, cid, mcc_i, cty, Wm, Wu, Wc, Wmcc, Wcty, out,
             iv0, iv1, iv2, iv3, iv4, b0, b1, b2, b3, b4, sem):
    wid = lax.axis_index("s") * NC + lax.axis_index("c")
    base = wid * PERW

    pltpu.sync_copy(mid.at[pl.ds(base, PERW)], iv0)
    pltpu.sync_copy(uid.at[pl.ds(base, PERW)], iv1)
    pltpu.sync_copy(cid.at[pl.ds(base, PERW)], iv2)
    pltpu.sync_copy(mcc_i.at[pl.ds(base, PERW)], iv3)
    pltpu.sync_copy(cty.at[pl.ds(base, PERW)], iv4)

    def step(g, carry):
        s = pl.ds(g * G, G)
        d0 = pltpu.async_copy(Wm.at[iv0.at[s]], b0, sem)
        d1 = pltpu.async_copy(Wu.at[iv1.at[s]], b1, sem)
        d2 = pltpu.async_copy(Wc.at[iv2.at[s]], b2, sem)
        d3 = pltpu.async_copy(Wmcc.at[iv3.at[s]], b3, sem)
        d4 = pltpu.async_copy(Wcty.at[iv4.at[s]], b4, sem)
        d0.wait(); d1.wait(); d2.wait(); d3.wait(); d4.wait()
        r = pl.ds(base + g * G, G)
        pltpu.sync_copy(b0, out.at[r, pl.ds(OFFS[0], DIMS[0])])
        pltpu.sync_copy(b1, out.at[r, pl.ds(OFFS[1], DIMS[1])])
        pltpu.sync_copy(b2, out.at[r, pl.ds(OFFS[2], DIMS[2])])
        pltpu.sync_copy(b3, out.at[r, pl.ds(OFFS[3], DIMS[3])])
        pltpu.sync_copy(b4, out.at[r, pl.ds(OFFS[4], DIMS[4])])
        return carry

    lax.fori_loop(0, NG, step, 0)


@jax.jit
def kernel(merchant_id, user_id, category_id, mcc, country,
           W_merchant_id, W_user_id, W_category_id, W_mcc, W_country):
    mesh = plsc.VectorSubcoreMesh(core_axis_name="c", subcore_axis_name="s")
    run = pl.kernel(
        _sc_body,
        out_type=jax.ShapeDtypeStruct((ROWS, DTOT), jnp.float32),
        mesh=mesh,
        scratch_types=(
            [pltpu.VMEM((PERW,), jnp.int32) for _ in range(5)]
            + [pltpu.VMEM((G, d), jnp.float32) for d in DIMS]
            + [pltpu.SemaphoreType.DMA]
        ),
        compiler_params=pltpu.CompilerParams(use_tc_tiling_on_sc=False),
    )
    out = run(
        merchant_id.reshape(ROWS), user_id.reshape(ROWS),
        category_id.reshape(ROWS), mcc.reshape(ROWS), country.reshape(ROWS),
        W_merchant_id, W_user_id, W_category_id, W_mcc, W_country,
    )
    return out.reshape(B, L, DTOT)


# speedup vs baseline: 4.2290x; 1.0515x over previous
import jax
import jax.numpy as jnp
from jax import lax
from jax.experimental import pallas as pl
from jax.experimental.pallas import tpu as pltpu
from jax.experimental.pallas import tpu_sc as plsc

B, L = 4096, 50
ROWS = B * L
G = 128
DIMS = (32, 32, 16, 16, 16)
OFFS = (0, 32, 64, 80, 96)
DTOT = 112

_info = plsc.get_sparse_core_info()
NC, NS = _info.num_cores, _info.num_subcores
NW = NC * NS
PERW = ROWS // NW
NG = PERW // G


def _sc_body(mid, uid, cid, mcc_i, cty, Wm, Wu, Wc, Wmcc, Wcty, out,
             iv0, iv1, iv2, iv3, iv4, bufsA, bufsB, gsemA, gsemB,
             wsemA, wsemB):
    wid = lax.axis_index("s") * NC + lax.axis_index("c")
    base = wid * PERW

    pltpu.sync_copy(mid.at[pl.ds(base, PERW)], iv0)
    pltpu.sync_copy(uid.at[pl.ds(base, PERW)], iv1)
    pltpu.sync_copy(cid.at[pl.ds(base, PERW)], iv2)
    pltpu.sync_copy(mcc_i.at[pl.ds(base, PERW)], iv3)
    pltpu.sync_copy(cty.at[pl.ds(base, PERW)], iv4)

    tabs = (Wm, Wu, Wc, Wmcc, Wcty)
    ivs = (iv0, iv1, iv2, iv3, iv4)

    def gathers(g, bufs, gsem):
        s = pl.ds(g * G, G)
        for t in range(5):
            pltpu.async_copy(tabs[t].at[ivs[t].at[s]], bufs[t], gsem)

    def wait_gathers(g, bufs, gsem):
        s = pl.ds(g * G, G)
        for t in range(5):
            pltpu.make_async_copy(tabs[t].at[ivs[t].at[s]], bufs[t], gsem).wait()

    def writes(g, bufs, wsem):
        r = pl.ds(base + g * G, G)
        for t in range(5):
            pltpu.async_copy(bufs[t], out.at[r, pl.ds(OFFS[t], DIMS[t])], wsem)

    def wait_writes(g, bufs, wsem):
        r = pl.ds(base + g * G, G)
        for t in range(5):
            pltpu.make_async_copy(bufs[t], out.at[r, pl.ds(OFFS[t], DIMS[t])],
                                  wsem).wait()

    def step(k, carry):
        g0, g1 = 2 * k, 2 * k + 1

        @pl.when(k > 0)
        def _():
            wait_writes(g0, bufsA, wsemA)
        gathers(g0, bufsA, gsemA)

        @pl.when(k > 0)
        def _():
            wait_writes(g1, bufsB, wsemB)
        gathers(g1, bufsB, gsemB)

        wait_gathers(g0, bufsA, gsemA)
        writes(g0, bufsA, wsemA)
        wait_gathers(g1, bufsB, gsemB)
        writes(g1, bufsB, wsemB)
        return carry

    lax.fori_loop(0, NG // 2, step, 0)
    wait_writes(NG - 2, bufsA, wsemA)
    wait_writes(NG - 1, bufsB, wsemB)


@jax.jit
def kernel(merchant_id, user_id, category_id, mcc, country,
           W_merchant_id, W_user_id, W_category_id, W_mcc, W_country):
    mesh = plsc.VectorSubcoreMesh(core_axis_name="c", subcore_axis_name="s")
    bufset = tuple(pltpu.VMEM((G, d), jnp.float32) for d in DIMS)
    run = pl.kernel(
        _sc_body,
        out_type=jax.ShapeDtypeStruct((ROWS, DTOT), jnp.float32),
        mesh=mesh,
        scratch_types=(
            [pltpu.VMEM((PERW,), jnp.int32) for _ in range(5)]
            + [bufset, bufset]
            + [pltpu.SemaphoreType.DMA] * 4
        ),
        compiler_params=pltpu.CompilerParams(use_tc_tiling_on_sc=False),
    )
    out = run(
        merchant_id.reshape(ROWS), user_id.reshape(ROWS),
        category_id.reshape(ROWS), mcc.reshape(ROWS), country.reshape(ROWS),
        W_merchant_id, W_user_id, W_category_id, W_mcc, W_country,
    )
    return out.reshape(B, L, DTOT)
